# initial kernel scaffold (unmeasured)
import jax
import jax.numpy as jnp
from jax import lax
from jax.experimental import pallas as pl
from jax.experimental.pallas import tpu as pltpu

N_DEV = 4
SQ = 1024
H_PER = 8
DH = 128
WIN = 128
KV_USED = SQ + WIN
SCALE = 0.08838834764831843
QBLK = 256


def kernel(x, Wq, K_ext, V_ext, Wo):
    def body(x_ref, wq_ref, k_ref, v_ref, wo_ref, out_ref,
             kbuf, vbuf, q_scr, comm,
             send_sems, recv_sems, local_sems, ar_send, ar_recv):
        i = lax.axis_index("i")
        right = lax.rem(i + 1, N_DEV)

        barrier = pltpu.get_barrier_semaphore()
        for off in (1, 2, 3):
            pl.semaphore_signal(
                barrier, inc=1,
                device_id=(lax.rem(i + off, N_DEV),),
                device_id_type=pl.DeviceIdType.MESH)
        pl.semaphore_wait(barrier, N_DEV - 1)

        s0_sends = []
        s1_sends = []
        for n, j in enumerate((1, 2, 3)):
            s0_sends.append(pltpu.make_async_remote_copy(
                src_ref=k_ref.at[0, :, 8 * j:8 * (j + 1), :],
                dst_ref=kbuf.at[0:SQ],
                send_sem=send_sems.at[2 * n],
                recv_sem=recv_sems.at[0],
                device_id=(j,), device_id_type=pl.DeviceIdType.MESH))
            s0_sends.append(pltpu.make_async_remote_copy(
                src_ref=v_ref.at[0, :, 8 * j:8 * (j + 1), :],
                dst_ref=vbuf.at[0:SQ],
                send_sem=send_sems.at[2 * n + 1],
                recv_sem=recv_sems.at[1],
                device_id=(j,), device_id_type=pl.DeviceIdType.MESH))
        for n, j in enumerate((0, 2, 3)):
            s1_sends.append(pltpu.make_async_remote_copy(
                src_ref=k_ref.at[0, 0:WIN, 8 * j:8 * (j + 1), :],
                dst_ref=kbuf.at[SQ:KV_USED],
                send_sem=send_sems.at[2 * n],
                recv_sem=recv_sems.at[2],
                device_id=(j,), device_id_type=pl.DeviceIdType.MESH))
            s1_sends.append(pltpu.make_async_remote_copy(
                src_ref=v_ref.at[0, 0:WIN, 8 * j:8 * (j + 1), :],
                dst_ref=vbuf.at[SQ:KV_USED],
                send_sem=send_sems.at[2 * n + 1],
                recv_sem=recv_sems.at[3],
                device_id=(j,), device_id_type=pl.DeviceIdType.MESH))

        lk0 = pltpu.make_async_copy(
            k_ref.at[0, :, 0:H_PER, :], kbuf.at[0:SQ], local_sems.at[0])
        lv0 = pltpu.make_async_copy(
            v_ref.at[0, :, 0:H_PER, :], vbuf.at[0:SQ], local_sems.at[1])
        lk1 = pltpu.make_async_copy(
            k_ref.at[0, 0:WIN, H_PER:2 * H_PER, :], kbuf.at[SQ:KV_USED],
            local_sems.at[2])
        lv1 = pltpu.make_async_copy(
            v_ref.at[0, 0:WIN, H_PER:2 * H_PER, :], vbuf.at[SQ:KV_USED],
            local_sems.at[3])

        r_k0 = pltpu.make_async_remote_copy(
            src_ref=kbuf.at[0:SQ], dst_ref=kbuf.at[0:SQ],
            send_sem=send_sems.at[0], recv_sem=recv_sems.at[0],
            device_id=(0,), device_id_type=pl.DeviceIdType.MESH)
        r_v0 = pltpu.make_async_remote_copy(
            src_ref=vbuf.at[0:SQ], dst_ref=vbuf.at[0:SQ],
            send_sem=send_sems.at[1], recv_sem=recv_sems.at[1],
            device_id=(0,), device_id_type=pl.DeviceIdType.MESH)
        r_k1 = pltpu.make_async_remote_copy(
            src_ref=kbuf.at[SQ:KV_USED], dst_ref=kbuf.at[SQ:KV_USED],
            send_sem=send_sems.at[2], recv_sem=recv_sems.at[2],
            device_id=(1,), device_id_type=pl.DeviceIdType.MESH)
        r_v1 = pltpu.make_async_remote_copy(
            src_ref=vbuf.at[SQ:KV_USED], dst_ref=vbuf.at[SQ:KV_USED],
            send_sem=send_sems.at[3], recv_sem=recv_sems.at[3],
            device_id=(1,), device_id_type=pl.DeviceIdType.MESH)

        @pl.when(i == 0)
        def _():
            lk0.start()
            lv0.start()
            for s in s0_sends:
                s.start()

        @pl.when(i == 1)
        def _():
            lk1.start()
            lv1.start()
            for s in s1_sends:
                s.start()

        q_scr[...] = jnp.dot(
            x_ref[0], wq_ref[...],
            preferred_element_type=jnp.float32) * SCALE

        @pl.when(i == 0)
        def _():
            lk0.wait()
            lv0.wait()

        @pl.when(i != 0)
        def _():
            r_k0.wait_recv()
            r_v0.wait_recv()

        @pl.when(i == 1)
        def _():
            lk1.wait()
            lv1.wait()

        @pl.when(i != 1)
        def _():
            r_k1.wait_recv()
            r_v1.wait_recv()

        for qb in range(SQ // QBLK):
            r0 = qb * QBLK
            ks = max(0, r0 - WIN)
            ke = min(KV_USED, r0 + QBLK + WIN)
            kw = ke - ks
            rows = r0 + lax.broadcasted_iota(jnp.int32, (QBLK, kw), 0)
            cols = ks + lax.broadcasted_iota(jnp.int32, (QBLK, kw), 1)
            mask = jnp.abs(rows - cols) <= WIN
            acc = jnp.zeros((QBLK, SQ), jnp.float32)
            for h in range(H_PER):
                qh = q_scr[r0:r0 + QBLK, h * DH:(h + 1) * DH]
                kh = kbuf[ks:ke, h, :]
                vh = vbuf[ks:ke, h, :]
                s = lax.dot_general(
                    qh, kh, (((1,), (1,)), ((), ())),
                    preferred_element_type=jnp.float32)
                s = jnp.where(mask, s, -1e9)
                m = jnp.max(s, axis=1, keepdims=True)
                e = jnp.exp(s - m)
                den = jnp.sum(e, axis=1, keepdims=True)
                ctx = lax.dot_general(
                    e, vh, (((1,), (0,)), ((), ())),
                    preferred_element_type=jnp.float32) / den
                acc = acc + lax.dot_general(
                    ctx, wo_ref[h * DH:(h + 1) * DH, :],
                    (((1,), (0,)), ((), ())),
                    preferred_element_type=jnp.float32)
            out_ref[0, r0:r0 + QBLK, :] = acc
            comm[0, r0:r0 + QBLK, :] = acc

        @pl.when(i == 0)
        def _():
            for s in s0_sends:
                s.wait_send()

        @pl.when(i == 1)
        def _():
            for s in s1_sends:
                s.wait_send()

        for h in range(N_DEV - 1):
            rdma = pltpu.make_async_remote_copy(
                src_ref=comm.at[h], dst_ref=comm.at[h + 1],
                send_sem=ar_send.at[h], recv_sem=ar_recv.at[h],
                device_id=(right,), device_id_type=pl.DeviceIdType.MESH)
            rdma.start()
            rdma.wait()
            out_ref[0, :, :] = out_ref[0, :, :] + comm[h + 1, :, :]

    return pl.pallas_call(
        body,
        out_shape=jax.ShapeDtypeStruct((1, SQ, SQ), jnp.float32),
        in_specs=[
            pl.BlockSpec(memory_space=pltpu.VMEM),
            pl.BlockSpec(memory_space=pltpu.VMEM),
            pl.BlockSpec(memory_space=pltpu.ANY),
            pl.BlockSpec(memory_space=pltpu.ANY),
            pl.BlockSpec(memory_space=pltpu.VMEM),
        ],
        out_specs=pl.BlockSpec(memory_space=pltpu.VMEM),
        scratch_shapes=[
            pltpu.VMEM((KV_USED, H_PER, DH), jnp.float32),
            pltpu.VMEM((KV_USED, H_PER, DH), jnp.float32),
            pltpu.VMEM((SQ, SQ), jnp.float32),
            pltpu.VMEM((N_DEV, SQ, SQ), jnp.float32),
            pltpu.SemaphoreType.DMA((6,)),
            pltpu.SemaphoreType.DMA((4,)),
            pltpu.SemaphoreType.DMA((4,)),
            pltpu.SemaphoreType.DMA((3,)),
            pltpu.SemaphoreType.DMA((3,)),
        ],
        compiler_params=pltpu.CompilerParams(collective_id=0),
    )(x, Wq, K_ext, V_ext, Wo)


# baseline (device time: 356581 ns/iter reference)
import jax
import jax.numpy as jnp
from jax import lax
from jax.experimental import pallas as pl
from jax.experimental.pallas import tpu as pltpu

N_DEV = 4
SQ = 1024
H_PER = 8
DH = 128
WIN = 128
KV_USED = SQ + WIN
SCALE = 0.08838834764831843
QBLK = 256


def kernel(x, Wq, K_ext, V_ext, Wo):
    def body(x_ref, wq_ref, k_ref, v_ref, wo_ref, out_ref,
             kbuf, vbuf, q_scr, comm,
             send_sems, recv_sems, local_sems, ar_send, ar_recv):
        i = lax.axis_index("i")
        right = lax.rem(i + 1, N_DEV)

        barrier = pltpu.get_barrier_semaphore()
        for off in (1, 2, 3):
            pl.semaphore_signal(
                barrier, inc=1,
                device_id=(lax.rem(i + off, N_DEV),),
                device_id_type=pl.DeviceIdType.MESH)
        pl.semaphore_wait(barrier, N_DEV - 1)

        s0_sends = []
        s1_sends = []
        for n, j in enumerate((1, 2, 3)):
            s0_sends.append(pltpu.make_async_remote_copy(
                src_ref=k_ref.at[0, :, 8 * j:8 * (j + 1), :],
                dst_ref=kbuf.at[0:SQ],
                send_sem=send_sems.at[2 * n],
                recv_sem=recv_sems.at[0],
                device_id=(j,), device_id_type=pl.DeviceIdType.MESH))
            s0_sends.append(pltpu.make_async_remote_copy(
                src_ref=v_ref.at[0, :, 8 * j:8 * (j + 1), :],
                dst_ref=vbuf.at[0:SQ],
                send_sem=send_sems.at[2 * n + 1],
                recv_sem=recv_sems.at[1],
                device_id=(j,), device_id_type=pl.DeviceIdType.MESH))
        for n, j in enumerate((0, 2, 3)):
            s1_sends.append(pltpu.make_async_remote_copy(
                src_ref=k_ref.at[0, 0:WIN, 8 * j:8 * (j + 1), :],
                dst_ref=kbuf.at[SQ:KV_USED],
                send_sem=send_sems.at[2 * n],
                recv_sem=recv_sems.at[2],
                device_id=(j,), device_id_type=pl.DeviceIdType.MESH))
            s1_sends.append(pltpu.make_async_remote_copy(
                src_ref=v_ref.at[0, 0:WIN, 8 * j:8 * (j + 1), :],
                dst_ref=vbuf.at[SQ:KV_USED],
                send_sem=send_sems.at[2 * n + 1],
                recv_sem=recv_sems.at[3],
                device_id=(j,), device_id_type=pl.DeviceIdType.MESH))

        lk0 = pltpu.make_async_copy(
            k_ref.at[0, :, 0:H_PER, :], kbuf.at[0:SQ], local_sems.at[0])
        lv0 = pltpu.make_async_copy(
            v_ref.at[0, :, 0:H_PER, :], vbuf.at[0:SQ], local_sems.at[1])
        lk1 = pltpu.make_async_copy(
            k_ref.at[0, 0:WIN, H_PER:2 * H_PER, :], kbuf.at[SQ:KV_USED],
            local_sems.at[2])
        lv1 = pltpu.make_async_copy(
            v_ref.at[0, 0:WIN, H_PER:2 * H_PER, :], vbuf.at[SQ:KV_USED],
            local_sems.at[3])

        r_k0 = pltpu.make_async_remote_copy(
            src_ref=kbuf.at[0:SQ], dst_ref=kbuf.at[0:SQ],
            send_sem=send_sems.at[0], recv_sem=recv_sems.at[0],
            device_id=(0,), device_id_type=pl.DeviceIdType.MESH)
        r_v0 = pltpu.make_async_remote_copy(
            src_ref=vbuf.at[0:SQ], dst_ref=vbuf.at[0:SQ],
            send_sem=send_sems.at[1], recv_sem=recv_sems.at[1],
            device_id=(0,), device_id_type=pl.DeviceIdType.MESH)
        r_k1 = pltpu.make_async_remote_copy(
            src_ref=kbuf.at[SQ:KV_USED], dst_ref=kbuf.at[SQ:KV_USED],
            send_sem=send_sems.at[2], recv_sem=recv_sems.at[2],
            device_id=(1,), device_id_type=pl.DeviceIdType.MESH)
        r_v1 = pltpu.make_async_remote_copy(
            src_ref=vbuf.at[SQ:KV_USED], dst_ref=vbuf.at[SQ:KV_USED],
            send_sem=send_sems.at[3], recv_sem=recv_sems.at[3],
            device_id=(1,), device_id_type=pl.DeviceIdType.MESH)

        @pl.when(i == 0)
        def _():
            lk0.start()
            lv0.start()
            for s in s0_sends:
                s.start()

        @pl.when(i == 1)
        def _():
            lk1.start()
            lv1.start()
            for s in s1_sends:
                s.start()

        q_scr[...] = jnp.dot(
            x_ref[0], wq_ref[...],
            preferred_element_type=jnp.float32) * SCALE

        @pl.when(i == 0)
        def _():
            lk0.wait()
            lv0.wait()

        @pl.when(i != 0)
        def _():
            r_k0.wait_recv()
            r_v0.wait_recv()

        @pl.when(i == 1)
        def _():
            lk1.wait()
            lv1.wait()

        @pl.when(i != 1)
        def _():
            r_k1.wait_recv()
            r_v1.wait_recv()

        for qb in range(SQ // QBLK):
            r0 = qb * QBLK
            ks = max(0, r0 - WIN)
            ke = min(KV_USED, r0 + QBLK + WIN)
            kw = ke - ks
            rows = r0 + lax.broadcasted_iota(jnp.int32, (QBLK, kw), 0)
            cols = ks + lax.broadcasted_iota(jnp.int32, (QBLK, kw), 1)
            mask = jnp.abs(rows - cols) <= WIN
            acc = jnp.zeros((QBLK, SQ), jnp.float32)
            for h in range(H_PER):
                qh = q_scr[r0:r0 + QBLK, h * DH:(h + 1) * DH]
                kh = kbuf[ks:ke, h, :]
                vh = vbuf[ks:ke, h, :]
                s = lax.dot_general(
                    qh, kh, (((1,), (1,)), ((), ())),
                    preferred_element_type=jnp.float32)
                s = jnp.where(mask, s, -1e9)
                m = jnp.max(s, axis=1, keepdims=True)
                e = jnp.exp(s - m)
                den = jnp.sum(e, axis=1, keepdims=True)
                ctx = lax.dot_general(
                    e, vh, (((1,), (0,)), ((), ())),
                    preferred_element_type=jnp.float32) / den
                acc = acc + lax.dot_general(
                    ctx, wo_ref[h * DH:(h + 1) * DH, :],
                    (((1,), (0,)), ((), ())),
                    preferred_element_type=jnp.float32)
            out_ref[0, r0:r0 + QBLK, :] = acc
            comm[0, r0:r0 + QBLK, :] = acc

        @pl.when(i == 0)
        def _():
            for s in s0_sends:
                s.wait_send()

        @pl.when(i == 1)
        def _():
            for s in s1_sends:
                s.wait_send()

        for h in range(N_DEV - 1):
            rdma = pltpu.make_async_remote_copy(
                src_ref=comm.at[h], dst_ref=comm.at[h + 1],
                send_sem=ar_send.at[h], recv_sem=ar_recv.at[h],
                device_id=(right,), device_id_type=pl.DeviceIdType.MESH)
            rdma.start()
            rdma.wait()
            out_ref[0, :, :] = out_ref[0, :, :] + comm[h + 1, :, :]

    return pl.pallas_call(
        body,
        out_shape=jax.ShapeDtypeStruct((1, SQ, SQ), jnp.float32),
        in_specs=[
            pl.BlockSpec(memory_space=pltpu.VMEM),
            pl.BlockSpec(memory_space=pltpu.VMEM),
            pl.BlockSpec(memory_space=pl.ANY),
            pl.BlockSpec(memory_space=pl.ANY),
            pl.BlockSpec(memory_space=pltpu.VMEM),
        ],
        out_specs=pl.BlockSpec(memory_space=pltpu.VMEM),
        scratch_shapes=[
            pltpu.VMEM((KV_USED, H_PER, DH), jnp.float32),
            pltpu.VMEM((KV_USED, H_PER, DH), jnp.float32),
            pltpu.VMEM((SQ, SQ), jnp.float32),
            pltpu.VMEM((N_DEV, SQ, SQ), jnp.float32),
            pltpu.SemaphoreType.DMA((6,)),
            pltpu.SemaphoreType.DMA((4,)),
            pltpu.SemaphoreType.DMA((4,)),
            pltpu.SemaphoreType.DMA((3,)),
            pltpu.SemaphoreType.DMA((3,)),
        ],
        compiler_params=pltpu.CompilerParams(collective_id=0),
    )(x, Wq, K_ext, V_ext, Wo)


# device time: 185990 ns/iter; 1.9172x vs baseline; 1.9172x over previous
import jax
import jax.numpy as jnp
from jax import lax
from jax.experimental import pallas as pl
from jax.experimental.pallas import tpu as pltpu

N_DEV = 4
SQ = 1024
H_PER = 8
NH = 32
DH = 128
WIN = 128
KV_USED = SQ + WIN
SCALE = 0.08838834764831843
QBLK = 256
BF = jnp.bfloat16


def kernel(x, Wq, K_ext, V_ext, Wo):
    def body(x_ref, wq_ref, k_ref, v_ref, wo_ref, out_ref,
             kbuf, vbuf, q_scr, wo16, tmp, kstage, vstage,
             rs_stage, rs_buf, ag_buf,
             send_sems, recv_sems, local_sems,
             rs_send, rs_recv, ag_send, ag_recv):
        i = lax.axis_index("i")
        right = lax.rem(i + 1, N_DEV)

        barrier = pltpu.get_barrier_semaphore()
        for off in (1, 2, 3):
            pl.semaphore_signal(
                barrier, inc=1,
                device_id=(lax.rem(i + off, N_DEV),),
                device_id_type=pl.DeviceIdType.MESH)
        pl.semaphore_wait(barrier, N_DEV - 1)

        s0_sends = []
        s1_sends = []
        for n, j in enumerate((1, 2, 3)):
            s0_sends.append(pltpu.make_async_remote_copy(
                src_ref=kstage.at[:, 8 * j:8 * (j + 1), :],
                dst_ref=kbuf.at[0:SQ],
                send_sem=send_sems.at[2 * n],
                recv_sem=recv_sems.at[0],
                device_id=(j,), device_id_type=pl.DeviceIdType.MESH))
            s0_sends.append(pltpu.make_async_remote_copy(
                src_ref=vstage.at[:, 8 * j:8 * (j + 1), :],
                dst_ref=vbuf.at[0:SQ],
                send_sem=send_sems.at[2 * n + 1],
                recv_sem=recv_sems.at[1],
                device_id=(j,), device_id_type=pl.DeviceIdType.MESH))
        for n, j in enumerate((0, 2, 3)):
            s1_sends.append(pltpu.make_async_remote_copy(
                src_ref=kstage.at[0:WIN, 8 * j:8 * (j + 1), :],
                dst_ref=kbuf.at[SQ:KV_USED],
                send_sem=send_sems.at[2 * n],
                recv_sem=recv_sems.at[2],
                device_id=(j,), device_id_type=pl.DeviceIdType.MESH))
            s1_sends.append(pltpu.make_async_remote_copy(
                src_ref=vstage.at[0:WIN, 8 * j:8 * (j + 1), :],
                dst_ref=vbuf.at[SQ:KV_USED],
                send_sem=send_sems.at[2 * n + 1],
                recv_sem=recv_sems.at[3],
                device_id=(j,), device_id_type=pl.DeviceIdType.MESH))

        r_k0 = pltpu.make_async_remote_copy(
            src_ref=kbuf.at[0:SQ], dst_ref=kbuf.at[0:SQ],
            send_sem=send_sems.at[0], recv_sem=recv_sems.at[0],
            device_id=(0,), device_id_type=pl.DeviceIdType.MESH)
        r_v0 = pltpu.make_async_remote_copy(
            src_ref=vbuf.at[0:SQ], dst_ref=vbuf.at[0:SQ],
            send_sem=send_sems.at[1], recv_sem=recv_sems.at[1],
            device_id=(0,), device_id_type=pl.DeviceIdType.MESH)
        r_k1 = pltpu.make_async_remote_copy(
            src_ref=kbuf.at[SQ:KV_USED], dst_ref=kbuf.at[SQ:KV_USED],
            send_sem=send_sems.at[2], recv_sem=recv_sems.at[2],
            device_id=(1,), device_id_type=pl.DeviceIdType.MESH)
        r_v1 = pltpu.make_async_remote_copy(
            src_ref=vbuf.at[SQ:KV_USED], dst_ref=vbuf.at[SQ:KV_USED],
            send_sem=send_sems.at[3], recv_sem=recv_sems.at[3],
            device_id=(1,), device_id_type=pl.DeviceIdType.MESH)

        @pl.when(i == 0)
        def _():
            h1 = pltpu.make_async_copy(
                k_ref.at[0, 0:SQ // 2], tmp, local_sems.at[0])
            h1.start()
            h1.wait()
            kstage[0:SQ // 2] = tmp[...].astype(BF)
            h2 = pltpu.make_async_copy(
                k_ref.at[0, SQ // 2:SQ], tmp, local_sems.at[1])
            h2.start()
            h2.wait()
            kstage[SQ // 2:SQ] = tmp[...].astype(BF)
            for s in s0_sends[0::2]:
                s.start()
            kbuf[0:SQ] = kstage[:, 0:H_PER, :]
            h3 = pltpu.make_async_copy(
                v_ref.at[0, 0:SQ // 2], tmp, local_sems.at[2])
            h3.start()
            h3.wait()
            vstage[0:SQ // 2] = tmp[...].astype(BF)
            h4 = pltpu.make_async_copy(
                v_ref.at[0, SQ // 2:SQ], tmp, local_sems.at[3])
            h4.start()
            h4.wait()
            vstage[SQ // 2:SQ] = tmp[...].astype(BF)
            for s in s0_sends[1::2]:
                s.start()
            vbuf[0:SQ] = vstage[:, 0:H_PER, :]

        @pl.when(i == 1)
        def _():
            h1 = pltpu.make_async_copy(
                k_ref.at[0, 0:WIN], tmp.at[0:WIN], local_sems.at[0])
            h1.start()
            h1.wait()
            kstage[0:WIN] = tmp[0:WIN].astype(BF)
            h2 = pltpu.make_async_copy(
                v_ref.at[0, 0:WIN], tmp.at[WIN:2 * WIN], local_sems.at[1])
            h2.start()
            h2.wait()
            vstage[0:WIN] = tmp[WIN:2 * WIN].astype(BF)
            for s in s1_sends:
                s.start()
            kbuf[SQ:KV_USED] = kstage[0:WIN, H_PER:2 * H_PER, :]
            vbuf[SQ:KV_USED] = vstage[0:WIN, H_PER:2 * H_PER, :]

        q_scr[...] = (jnp.dot(
            x_ref[0].astype(BF), wq_ref[...].astype(BF),
            preferred_element_type=jnp.float32) * SCALE).astype(BF)
        wo16[...] = wo_ref[...].astype(BF)

        @pl.when(i != 0)
        def _():
            r_k0.wait_recv()
            r_v0.wait_recv()

        @pl.when(i != 1)
        def _():
            r_k1.wait_recv()
            r_v1.wait_recv()

        for qb in range(SQ // QBLK):
            r0 = qb * QBLK
            ks = max(0, r0 - WIN)
            ke = min(KV_USED, r0 + QBLK + WIN)
            kw = ke - ks
            rows = r0 + lax.broadcasted_iota(jnp.int32, (QBLK, kw), 0)
            cols = ks + lax.broadcasted_iota(jnp.int32, (QBLK, kw), 1)
            mask = jnp.abs(rows - cols) <= WIN
            acc = jnp.zeros((QBLK, SQ), jnp.float32)
            for h in range(H_PER):
                qh = q_scr[r0:r0 + QBLK, h * DH:(h + 1) * DH]
                kh = kbuf[ks:ke, h, :]
                vh = vbuf[ks:ke, h, :]
                s = lax.dot_general(
                    qh, kh, (((1,), (1,)), ((), ())),
                    preferred_element_type=jnp.float32)
                s = jnp.where(mask, s, -1e9)
                m = jnp.max(s, axis=1, keepdims=True)
                e = jnp.exp(s - m)
                den = jnp.sum(e, axis=1, keepdims=True)
                ctx = lax.dot_general(
                    e.astype(BF), vh, (((1,), (0,)), ((), ())),
                    preferred_element_type=jnp.float32) / den
                acc = acc + lax.dot_general(
                    ctx.astype(BF), wo16[h * DH:(h + 1) * DH, :],
                    (((1,), (0,)), ((), ())),
                    preferred_element_type=jnp.float32)
            out_ref[0, r0:r0 + QBLK, :] = acc

        @pl.when(i == 0)
        def _():
            for s in s0_sends:
                s.wait_send()

        @pl.when(i == 1)
        def _():
            for s in s1_sends:
                s.wait_send()

        cs0 = lax.rem(i + N_DEV, N_DEV)
        rs_stage[0] = out_ref[0, pl.ds(cs0 * QBLK, QBLK), :].astype(BF)
        for h in range(N_DEV - 1):
            rdma = pltpu.make_async_remote_copy(
                src_ref=rs_stage.at[h], dst_ref=rs_buf.at[h],
                send_sem=rs_send.at[h], recv_sem=rs_recv.at[h],
                device_id=(right,), device_id_type=pl.DeviceIdType.MESH)
            rdma.start()
            rdma.wait()
            cr = lax.rem(i - h - 1 + 2 * N_DEV, N_DEV)
            red = (out_ref[0, pl.ds(cr * QBLK, QBLK), :]
                   + rs_buf[h].astype(jnp.float32))
            if h < N_DEV - 2:
                rs_stage[h + 1] = red.astype(BF)
            else:
                out_ref[0, pl.ds(cr * QBLK, QBLK), :] = red
                ag_buf[cr] = red.astype(BF)

        for h in range(N_DEV - 1):
            s_c = lax.rem(i + 1 - h + 2 * N_DEV, N_DEV)
            r_c = lax.rem(i - h + 2 * N_DEV, N_DEV)
            rdma = pltpu.make_async_remote_copy(
                src_ref=ag_buf.at[s_c], dst_ref=ag_buf.at[s_c],
                send_sem=ag_send.at[h], recv_sem=ag_recv.at[h],
                device_id=(right,), device_id_type=pl.DeviceIdType.MESH)
            rdma.start()
            rdma.wait()
            out_ref[0, pl.ds(r_c * QBLK, QBLK), :] = (
                ag_buf[r_c].astype(jnp.float32))

    return pl.pallas_call(
        body,
        out_shape=jax.ShapeDtypeStruct((1, SQ, SQ), jnp.float32),
        in_specs=[
            pl.BlockSpec(memory_space=pltpu.VMEM),
            pl.BlockSpec(memory_space=pltpu.VMEM),
            pl.BlockSpec(memory_space=pl.ANY),
            pl.BlockSpec(memory_space=pl.ANY),
            pl.BlockSpec(memory_space=pltpu.VMEM),
        ],
        out_specs=pl.BlockSpec(memory_space=pltpu.VMEM),
        scratch_shapes=[
            pltpu.VMEM((KV_USED, H_PER, DH), BF),
            pltpu.VMEM((KV_USED, H_PER, DH), BF),
            pltpu.VMEM((SQ, SQ), BF),
            pltpu.VMEM((SQ, SQ), BF),
            pltpu.VMEM((SQ // 2, NH, DH), jnp.float32),
            pltpu.VMEM((SQ, NH, DH), BF),
            pltpu.VMEM((SQ, NH, DH), BF),
            pltpu.VMEM((N_DEV - 1, QBLK, SQ), BF),
            pltpu.VMEM((N_DEV - 1, QBLK, SQ), BF),
            pltpu.VMEM((N_DEV, QBLK, SQ), BF),
            pltpu.SemaphoreType.DMA((6,)),
            pltpu.SemaphoreType.DMA((4,)),
            pltpu.SemaphoreType.DMA((4,)),
            pltpu.SemaphoreType.DMA((3,)),
            pltpu.SemaphoreType.DMA((3,)),
            pltpu.SemaphoreType.DMA((3,)),
            pltpu.SemaphoreType.DMA((3,)),
        ],
        compiler_params=pltpu.CompilerParams(
            collective_id=0, vmem_limit_bytes=60 * 1024 * 1024),
    )(x, Wq, K_ext, V_ext, Wo)


# device time: 148329 ns/iter; 2.4040x vs baseline; 1.2539x over previous
import jax
import jax.numpy as jnp
from jax import lax
from jax.experimental import pallas as pl
from jax.experimental.pallas import tpu as pltpu

N_DEV = 4
SQ = 1024
H_PER = 8
NH = 32
DH = 128
WIN = 128
KV_USED = SQ + WIN
KW = 512
SCALE = 0.08838834764831843
QBLK = 256
BF = jnp.bfloat16
MESH = pl.DeviceIdType.MESH


def kernel(x, Wq, K_ext, V_ext, Wo):
    def body(x_ref, wq_ref, k_ref, v_ref, wo_ref, out_ref,
             kbuf, vbuf, q_scr, wo16, tmp, kstage, vstage, relay,
             rs_stage, rs_buf, ag_buf,
             send_sems, recv_sems, local_sems, relay_sems,
             rs_send, rs_recv, ag_send, ag_recv):
        i = lax.axis_index("i")
        right = lax.rem(i + 1, N_DEV)

        barrier = pltpu.get_barrier_semaphore()
        for off in (1, 2, 3):
            pl.semaphore_signal(
                barrier, inc=1,
                device_id=(lax.rem(i + off, N_DEV),),
                device_id_type=MESH)
        pl.semaphore_wait(barrier, N_DEV - 1)

        def head_slice(st, j):
            return st.at[:, 8 * j:8 * (j + 1), :]

        s0_k = [
            pltpu.make_async_remote_copy(
                src_ref=head_slice(kstage, 2), dst_ref=relay,
                send_sem=send_sems.at[0], recv_sem=relay_sems.at[0],
                device_id=(1,), device_id_type=MESH),
            pltpu.make_async_remote_copy(
                src_ref=head_slice(kstage, 1), dst_ref=kbuf.at[0:SQ],
                send_sem=send_sems.at[1], recv_sem=recv_sems.at[0],
                device_id=(1,), device_id_type=MESH),
            pltpu.make_async_remote_copy(
                src_ref=head_slice(kstage, 3), dst_ref=kbuf.at[0:SQ],
                send_sem=send_sems.at[2], recv_sem=recv_sems.at[0],
                device_id=(3,), device_id_type=MESH),
        ]
        s0_v = [
            pltpu.make_async_remote_copy(
                src_ref=head_slice(vstage, 2), dst_ref=relay,
                send_sem=send_sems.at[3], recv_sem=relay_sems.at[1],
                device_id=(3,), device_id_type=MESH),
            pltpu.make_async_remote_copy(
                src_ref=head_slice(vstage, 1), dst_ref=vbuf.at[0:SQ],
                send_sem=send_sems.at[4], recv_sem=recv_sems.at[1],
                device_id=(1,), device_id_type=MESH),
            pltpu.make_async_remote_copy(
                src_ref=head_slice(vstage, 3), dst_ref=vbuf.at[0:SQ],
                send_sem=send_sems.at[5], recv_sem=recv_sems.at[1],
                device_id=(3,), device_id_type=MESH),
        ]
        fwd_k = pltpu.make_async_remote_copy(
            src_ref=relay, dst_ref=kbuf.at[0:SQ],
            send_sem=send_sems.at[6], recv_sem=recv_sems.at[0],
            device_id=(2,), device_id_type=MESH)
        fwd_v = pltpu.make_async_remote_copy(
            src_ref=relay, dst_ref=vbuf.at[0:SQ],
            send_sem=send_sems.at[6], recv_sem=recv_sems.at[1],
            device_id=(2,), device_id_type=MESH)
        r_relay_k = pltpu.make_async_remote_copy(
            src_ref=relay, dst_ref=relay,
            send_sem=send_sems.at[0], recv_sem=relay_sems.at[0],
            device_id=(0,), device_id_type=MESH)
        r_relay_v = pltpu.make_async_remote_copy(
            src_ref=relay, dst_ref=relay,
            send_sem=send_sems.at[0], recv_sem=relay_sems.at[1],
            device_id=(0,), device_id_type=MESH)

        s1_sends = []
        for n, j in enumerate((0, 2, 3)):
            s1_sends.append(pltpu.make_async_remote_copy(
                src_ref=kstage.at[0:WIN, 8 * j:8 * (j + 1), :],
                dst_ref=kbuf.at[SQ:KV_USED],
                send_sem=send_sems.at[n],
                recv_sem=recv_sems.at[2],
                device_id=(j,), device_id_type=MESH))
            s1_sends.append(pltpu.make_async_remote_copy(
                src_ref=vstage.at[0:WIN, 8 * j:8 * (j + 1), :],
                dst_ref=vbuf.at[SQ:KV_USED],
                send_sem=send_sems.at[3 + n],
                recv_sem=recv_sems.at[3],
                device_id=(j,), device_id_type=MESH))

        r_k0 = pltpu.make_async_remote_copy(
            src_ref=kbuf.at[0:SQ], dst_ref=kbuf.at[0:SQ],
            send_sem=send_sems.at[7], recv_sem=recv_sems.at[0],
            device_id=(0,), device_id_type=MESH)
        r_v0 = pltpu.make_async_remote_copy(
            src_ref=vbuf.at[0:SQ], dst_ref=vbuf.at[0:SQ],
            send_sem=send_sems.at[7], recv_sem=recv_sems.at[1],
            device_id=(0,), device_id_type=MESH)
        r_k1 = pltpu.make_async_remote_copy(
            src_ref=kbuf.at[SQ:KV_USED], dst_ref=kbuf.at[SQ:KV_USED],
            send_sem=send_sems.at[7], recv_sem=recv_sems.at[2],
            device_id=(1,), device_id_type=MESH)
        r_v1 = pltpu.make_async_remote_copy(
            src_ref=vbuf.at[SQ:KV_USED], dst_ref=vbuf.at[SQ:KV_USED],
            send_sem=send_sems.at[7], recv_sem=recv_sems.at[3],
            device_id=(1,), device_id_type=MESH)

        QTR = SQ // 4

        @pl.when(i == 0)
        def _():
            for q in range(4):
                cp = pltpu.make_async_copy(
                    k_ref.at[0, q * QTR:(q + 1) * QTR], tmp,
                    local_sems.at[q % 2])
                cp.start()
                cp.wait()
                kstage[q * QTR:(q + 1) * QTR] = tmp[...].astype(BF)
            for s in s0_k:
                s.start()
            kbuf[0:SQ] = kstage[:, 0:H_PER, :]
            for q in range(4):
                cp = pltpu.make_async_copy(
                    v_ref.at[0, q * QTR:(q + 1) * QTR], tmp,
                    local_sems.at[2 + q % 2])
                cp.start()
                cp.wait()
                vstage[q * QTR:(q + 1) * QTR] = tmp[...].astype(BF)
            for s in s0_v:
                s.start()
            vbuf[0:SQ] = vstage[:, 0:H_PER, :]

        @pl.when(i == 1)
        def _():
            h1 = pltpu.make_async_copy(
                k_ref.at[0, 0:WIN], tmp.at[0:WIN], local_sems.at[0])
            h1.start()
            h1.wait()
            kstage[0:WIN] = tmp[0:WIN].astype(BF)
            h2 = pltpu.make_async_copy(
                v_ref.at[0, 0:WIN], tmp.at[WIN:2 * WIN], local_sems.at[1])
            h2.start()
            h2.wait()
            vstage[0:WIN] = tmp[WIN:2 * WIN].astype(BF)
            for s in s1_sends:
                s.start()
            kbuf[SQ:KV_USED] = kstage[0:WIN, H_PER:2 * H_PER, :]
            vbuf[SQ:KV_USED] = vstage[0:WIN, H_PER:2 * H_PER, :]

        q_scr[...] = (jnp.dot(
            x_ref[0].astype(BF), wq_ref[...].astype(BF),
            preferred_element_type=jnp.float32) * SCALE).astype(BF)
        wo16[...] = wo_ref[...].astype(BF)

        @pl.when(i == 1)
        def _():
            r_relay_k.wait_recv()
            fwd_k.start()

        @pl.when(i == 3)
        def _():
            r_relay_v.wait_recv()
            fwd_v.start()

        @pl.when(i != 0)
        def _():
            r_k0.wait_recv()
            r_v0.wait_recv()

        @pl.when(i != 1)
        def _():
            r_k1.wait_recv()
            r_v1.wait_recv()

        rs_rdmas = []
        for t in range(N_DEV):
            cb = lax.rem(i - t + 2 * N_DEV, N_DEV)
            r0 = cb * QBLK
            ks = jnp.minimum(jnp.maximum(r0 - WIN, 0), KV_USED - KW)
            rows = r0 + lax.broadcasted_iota(jnp.int32, (QBLK, KW), 0)
            cols = ks + lax.broadcasted_iota(jnp.int32, (QBLK, KW), 1)
            mask = jnp.abs(rows - cols) <= WIN
            acc = jnp.zeros((QBLK, SQ), jnp.float32)
            for h in range(H_PER):
                qh = q_scr[pl.ds(r0, QBLK), h * DH:(h + 1) * DH]
                kh = kbuf[pl.ds(ks, KW), h, :]
                vh = vbuf[pl.ds(ks, KW), h, :]
                s = lax.dot_general(
                    qh, kh, (((1,), (1,)), ((), ())),
                    preferred_element_type=jnp.float32)
                s = jnp.where(mask, s, -1e9)
                m = jnp.max(s, axis=1, keepdims=True)
                e = jnp.exp(s - m)
                den = jnp.sum(e, axis=1, keepdims=True)
                ctx = lax.dot_general(
                    e.astype(BF), vh, (((1,), (0,)), ((), ())),
                    preferred_element_type=jnp.float32) / den
                acc = acc + lax.dot_general(
                    ctx.astype(BF), wo16[h * DH:(h + 1) * DH, :],
                    (((1,), (0,)), ((), ())),
                    preferred_element_type=jnp.float32)
            out_ref[0, pl.ds(r0, QBLK), :] = acc

            if t == 0:
                rs_stage[0] = acc.astype(BF)
            else:
                rs_rdmas[t - 1].wait_recv()
                red = acc + rs_buf[t - 1].astype(jnp.float32)
                if t < N_DEV - 1:
                    rs_stage[t] = red.astype(BF)
                else:
                    out_ref[0, pl.ds(r0, QBLK), :] = red
                    ag_buf[cb] = red.astype(BF)
            if t < N_DEV - 1:
                rdma = pltpu.make_async_remote_copy(
                    src_ref=rs_stage.at[t], dst_ref=rs_buf.at[t],
                    send_sem=rs_send.at[t], recv_sem=rs_recv.at[t],
                    device_id=(right,), device_id_type=MESH)
                rdma.start()
                rs_rdmas.append(rdma)

        ag_rdmas = []
        for h in range(N_DEV - 1):
            s_c = lax.rem(i + 1 - h + 2 * N_DEV, N_DEV)
            r_c = lax.rem(i - h + 2 * N_DEV, N_DEV)
            rdma = pltpu.make_async_remote_copy(
                src_ref=ag_buf.at[s_c], dst_ref=ag_buf.at[s_c],
                send_sem=ag_send.at[h], recv_sem=ag_recv.at[h],
                device_id=(right,), device_id_type=MESH)
            rdma.start()
            rdma.wait_recv()
            out_ref[0, pl.ds(r_c * QBLK, QBLK), :] = (
                ag_buf[r_c].astype(jnp.float32))
            ag_rdmas.append(rdma)

        for r in rs_rdmas + ag_rdmas:
            r.wait_send()

        @pl.when(i == 0)
        def _():
            for s in s0_k + s0_v:
                s.wait_send()

        @pl.when(i == 1)
        def _():
            for s in s1_sends:
                s.wait_send()
            fwd_k.wait_send()

        @pl.when(i == 3)
        def _():
            fwd_v.wait_send()

    return pl.pallas_call(
        body,
        out_shape=jax.ShapeDtypeStruct((1, SQ, SQ), jnp.float32),
        in_specs=[
            pl.BlockSpec(memory_space=pltpu.VMEM),
            pl.BlockSpec(memory_space=pltpu.VMEM),
            pl.BlockSpec(memory_space=pl.ANY),
            pl.BlockSpec(memory_space=pl.ANY),
            pl.BlockSpec(memory_space=pltpu.VMEM),
        ],
        out_specs=pl.BlockSpec(memory_space=pltpu.VMEM),
        scratch_shapes=[
            pltpu.VMEM((KV_USED, H_PER, DH), BF),
            pltpu.VMEM((KV_USED, H_PER, DH), BF),
            pltpu.VMEM((SQ, SQ), BF),
            pltpu.VMEM((SQ, SQ), BF),
            pltpu.VMEM((SQ // 4, NH, DH), jnp.float32),
            pltpu.VMEM((SQ, NH, DH), BF),
            pltpu.VMEM((SQ, NH, DH), BF),
            pltpu.VMEM((SQ, H_PER, DH), BF),
            pltpu.VMEM((N_DEV - 1, QBLK, SQ), BF),
            pltpu.VMEM((N_DEV - 1, QBLK, SQ), BF),
            pltpu.VMEM((N_DEV, QBLK, SQ), BF),
            pltpu.SemaphoreType.DMA((8,)),
            pltpu.SemaphoreType.DMA((4,)),
            pltpu.SemaphoreType.DMA((4,)),
            pltpu.SemaphoreType.DMA((2,)),
            pltpu.SemaphoreType.DMA((3,)),
            pltpu.SemaphoreType.DMA((3,)),
            pltpu.SemaphoreType.DMA((3,)),
            pltpu.SemaphoreType.DMA((3,)),
        ],
        compiler_params=pltpu.CompilerParams(
            collective_id=0, vmem_limit_bytes=60 * 1024 * 1024),
    )(x, Wq, K_ext, V_ext, Wo)


# device time: 140817 ns/iter; 2.5322x vs baseline; 1.0533x over previous
import jax
import jax.numpy as jnp
from jax import lax
from jax.experimental import pallas as pl
from jax.experimental.pallas import tpu as pltpu

N_DEV = 4
SQ = 1024
H_PER = 8
NH = 32
DH = 128
WIN = 128
KV_USED = SQ + WIN
KW = 512
SCALE = 0.08838834764831843
QBLK = 256
BF = jnp.bfloat16
MESH = pl.DeviceIdType.MESH


def kernel(x, Wq, K_ext, V_ext, Wo):
    def body(x_ref, wq_ref, k_ref, v_ref, wo_ref, out_ref,
             kbuf, vbuf, q_scr, wo16, tmp, kstage, vstage, relay,
             rs_stage, rs_buf, ag_buf,
             send_sems, recv_sems, local_sems, relay_sems,
             rs_send, rs_recv, ag_send, ag_recv, ag2_send, ag2_recv):
        i = lax.axis_index("i")
        right = lax.rem(i + 1, N_DEV)

        barrier = pltpu.get_barrier_semaphore()
        for off in (1, 2, 3):
            pl.semaphore_signal(
                barrier, inc=1,
                device_id=(lax.rem(i + off, N_DEV),),
                device_id_type=MESH)
        pl.semaphore_wait(barrier, N_DEV - 1)

        def head_slice(st, j):
            return st.at[:, 8 * j:8 * (j + 1), :]

        s0_k = [
            pltpu.make_async_remote_copy(
                src_ref=head_slice(kstage, 2), dst_ref=relay,
                send_sem=send_sems.at[0], recv_sem=relay_sems.at[0],
                device_id=(1,), device_id_type=MESH),
            pltpu.make_async_remote_copy(
                src_ref=head_slice(kstage, 1), dst_ref=kbuf.at[0:SQ],
                send_sem=send_sems.at[1], recv_sem=recv_sems.at[0],
                device_id=(1,), device_id_type=MESH),
            pltpu.make_async_remote_copy(
                src_ref=head_slice(kstage, 3), dst_ref=kbuf.at[0:SQ],
                send_sem=send_sems.at[2], recv_sem=recv_sems.at[0],
                device_id=(3,), device_id_type=MESH),
        ]
        s0_v = [
            pltpu.make_async_remote_copy(
                src_ref=head_slice(vstage, 2), dst_ref=relay,
                send_sem=send_sems.at[3], recv_sem=relay_sems.at[1],
                device_id=(3,), device_id_type=MESH),
            pltpu.make_async_remote_copy(
                src_ref=head_slice(vstage, 1), dst_ref=vbuf.at[0:SQ],
                send_sem=send_sems.at[4], recv_sem=recv_sems.at[1],
                device_id=(1,), device_id_type=MESH),
            pltpu.make_async_remote_copy(
                src_ref=head_slice(vstage, 3), dst_ref=vbuf.at[0:SQ],
                send_sem=send_sems.at[5], recv_sem=recv_sems.at[1],
                device_id=(3,), device_id_type=MESH),
        ]
        fwd_k = pltpu.make_async_remote_copy(
            src_ref=relay, dst_ref=kbuf.at[0:SQ],
            send_sem=send_sems.at[6], recv_sem=recv_sems.at[0],
            device_id=(2,), device_id_type=MESH)
        fwd_v = pltpu.make_async_remote_copy(
            src_ref=relay, dst_ref=vbuf.at[0:SQ],
            send_sem=send_sems.at[6], recv_sem=recv_sems.at[1],
            device_id=(2,), device_id_type=MESH)
        r_relay_k = pltpu.make_async_remote_copy(
            src_ref=relay, dst_ref=relay,
            send_sem=send_sems.at[0], recv_sem=relay_sems.at[0],
            device_id=(0,), device_id_type=MESH)
        r_relay_v = pltpu.make_async_remote_copy(
            src_ref=relay, dst_ref=relay,
            send_sem=send_sems.at[0], recv_sem=relay_sems.at[1],
            device_id=(0,), device_id_type=MESH)

        s1_sends = []
        for n, j in enumerate((0, 2, 3)):
            s1_sends.append(pltpu.make_async_remote_copy(
                src_ref=kstage.at[0:WIN, 8 * j:8 * (j + 1), :],
                dst_ref=kbuf.at[SQ:KV_USED],
                send_sem=send_sems.at[n],
                recv_sem=recv_sems.at[2],
                device_id=(j,), device_id_type=MESH))
            s1_sends.append(pltpu.make_async_remote_copy(
                src_ref=vstage.at[0:WIN, 8 * j:8 * (j + 1), :],
                dst_ref=vbuf.at[SQ:KV_USED],
                send_sem=send_sems.at[3 + n],
                recv_sem=recv_sems.at[3],
                device_id=(j,), device_id_type=MESH))

        r_k0 = pltpu.make_async_remote_copy(
            src_ref=kbuf.at[0:SQ], dst_ref=kbuf.at[0:SQ],
            send_sem=send_sems.at[7], recv_sem=recv_sems.at[0],
            device_id=(0,), device_id_type=MESH)
        r_v0 = pltpu.make_async_remote_copy(
            src_ref=vbuf.at[0:SQ], dst_ref=vbuf.at[0:SQ],
            send_sem=send_sems.at[7], recv_sem=recv_sems.at[1],
            device_id=(0,), device_id_type=MESH)
        r_k1 = pltpu.make_async_remote_copy(
            src_ref=kbuf.at[SQ:KV_USED], dst_ref=kbuf.at[SQ:KV_USED],
            send_sem=send_sems.at[7], recv_sem=recv_sems.at[2],
            device_id=(1,), device_id_type=MESH)
        r_v1 = pltpu.make_async_remote_copy(
            src_ref=vbuf.at[SQ:KV_USED], dst_ref=vbuf.at[SQ:KV_USED],
            send_sem=send_sems.at[7], recv_sem=recv_sems.at[3],
            device_id=(1,), device_id_type=MESH)

        QTR = SQ // 4

        @pl.when(i == 0)
        def _():
            for q in range(4):
                cp = pltpu.make_async_copy(
                    k_ref.at[0, q * QTR:(q + 1) * QTR], tmp,
                    local_sems.at[q % 2])
                cp.start()
                cp.wait()
                kstage[q * QTR:(q + 1) * QTR] = tmp[...].astype(BF)
            for s in s0_k:
                s.start()
            kbuf[0:SQ] = kstage[:, 0:H_PER, :]
            for q in range(4):
                cp = pltpu.make_async_copy(
                    v_ref.at[0, q * QTR:(q + 1) * QTR], tmp,
                    local_sems.at[2 + q % 2])
                cp.start()
                cp.wait()
                vstage[q * QTR:(q + 1) * QTR] = tmp[...].astype(BF)
            for s in s0_v:
                s.start()
            vbuf[0:SQ] = vstage[:, 0:H_PER, :]

        @pl.when(i == 1)
        def _():
            h1 = pltpu.make_async_copy(
                k_ref.at[0, 0:WIN], tmp.at[0:WIN], local_sems.at[0])
            h1.start()
            h1.wait()
            kstage[0:WIN] = tmp[0:WIN].astype(BF)
            h2 = pltpu.make_async_copy(
                v_ref.at[0, 0:WIN], tmp.at[WIN:2 * WIN], local_sems.at[1])
            h2.start()
            h2.wait()
            vstage[0:WIN] = tmp[WIN:2 * WIN].astype(BF)
            for s in s1_sends:
                s.start()
            kbuf[SQ:KV_USED] = kstage[0:WIN, H_PER:2 * H_PER, :]
            vbuf[SQ:KV_USED] = vstage[0:WIN, H_PER:2 * H_PER, :]

        q_scr[...] = (jnp.dot(
            x_ref[0].astype(BF), wq_ref[...].astype(BF),
            preferred_element_type=jnp.float32) * SCALE).astype(BF)
        wo16[...] = wo_ref[...].astype(BF)

        @pl.when(i == 1)
        def _():
            r_relay_k.wait_recv()
            fwd_k.start()

        @pl.when(i == 3)
        def _():
            r_relay_v.wait_recv()
            fwd_v.start()

        @pl.when(i != 0)
        def _():
            r_k0.wait_recv()
            r_v0.wait_recv()

        @pl.when(i != 1)
        def _():
            r_k1.wait_recv()
            r_v1.wait_recv()

        rs_rdmas = []
        for t in range(N_DEV):
            cb = lax.rem(i - t + 2 * N_DEV, N_DEV)
            r0 = cb * QBLK
            ks = jnp.minimum(jnp.maximum(r0 - WIN, 0), KV_USED - KW)
            rows = r0 + lax.broadcasted_iota(jnp.int32, (QBLK, KW), 0)
            cols = ks + lax.broadcasted_iota(jnp.int32, (QBLK, KW), 1)
            mask = jnp.abs(rows - cols) <= WIN
            acc = jnp.zeros((QBLK, SQ), jnp.float32)
            for h in range(H_PER):
                qh = q_scr[pl.ds(r0, QBLK), h * DH:(h + 1) * DH]
                kh = kbuf[pl.ds(ks, KW), h, :]
                vh = vbuf[pl.ds(ks, KW), h, :]
                s = lax.dot_general(
                    qh, kh, (((1,), (1,)), ((), ())),
                    preferred_element_type=jnp.float32)
                s = jnp.where(mask, s, -1e9)
                m = jnp.max(s, axis=1, keepdims=True)
                e = jnp.exp(s - m)
                den = jnp.sum(e, axis=1, keepdims=True)
                ctx = lax.dot_general(
                    e.astype(BF), vh, (((1,), (0,)), ((), ())),
                    preferred_element_type=jnp.float32) / den
                acc = acc + lax.dot_general(
                    ctx.astype(BF), wo16[h * DH:(h + 1) * DH, :],
                    (((1,), (0,)), ((), ())),
                    preferred_element_type=jnp.float32)
            out_ref[0, pl.ds(r0, QBLK), :] = acc

            if t == 0:
                rs_stage[0] = acc.astype(BF)
            else:
                rs_rdmas[t - 1].wait_recv()
                red = acc + rs_buf[t - 1].astype(jnp.float32)
                if t < N_DEV - 1:
                    rs_stage[t] = red.astype(BF)
                else:
                    out_ref[0, pl.ds(r0, QBLK), :] = red
                    ag_buf[cb] = red.astype(BF)
            if t < N_DEV - 1:
                rdma = pltpu.make_async_remote_copy(
                    src_ref=rs_stage.at[t], dst_ref=rs_buf.at[t],
                    send_sem=rs_send.at[t], recv_sem=rs_recv.at[t],
                    device_id=(right,), device_id_type=MESH)
                rdma.start()
                rs_rdmas.append(rdma)

        left = lax.rem(i - 1 + N_DEV, N_DEV)
        HC = SQ // 2
        ag_rdmas = []
        for h in range(N_DEV - 1):
            cw_s = lax.rem(i + 1 - h + 2 * N_DEV, N_DEV)
            cw_r = lax.rem(i - h + 2 * N_DEV, N_DEV)
            ccw_s = lax.rem(i + 1 + h, N_DEV)
            ccw_r = lax.rem(i + 2 + h, N_DEV)
            cw = pltpu.make_async_remote_copy(
                src_ref=ag_buf.at[cw_s, :, 0:HC],
                dst_ref=ag_buf.at[cw_s, :, 0:HC],
                send_sem=ag_send.at[h], recv_sem=ag_recv.at[h],
                device_id=(right,), device_id_type=MESH)
            ccw = pltpu.make_async_remote_copy(
                src_ref=ag_buf.at[ccw_s, :, HC:SQ],
                dst_ref=ag_buf.at[ccw_s, :, HC:SQ],
                send_sem=ag2_send.at[h], recv_sem=ag2_recv.at[h],
                device_id=(left,), device_id_type=MESH)
            cw.start()
            ccw.start()
            cw.wait_recv()
            ccw.wait_recv()
            out_ref[0, pl.ds(cw_r * QBLK, QBLK), 0:HC] = (
                ag_buf[cw_r, :, 0:HC].astype(jnp.float32))
            out_ref[0, pl.ds(ccw_r * QBLK, QBLK), HC:SQ] = (
                ag_buf[ccw_r, :, HC:SQ].astype(jnp.float32))
            ag_rdmas.append(cw)
            ag_rdmas.append(ccw)

        for r in rs_rdmas + ag_rdmas:
            r.wait_send()

        @pl.when(i == 0)
        def _():
            for s in s0_k + s0_v:
                s.wait_send()

        @pl.when(i == 1)
        def _():
            for s in s1_sends:
                s.wait_send()
            fwd_k.wait_send()

        @pl.when(i == 3)
        def _():
            fwd_v.wait_send()

    return pl.pallas_call(
        body,
        out_shape=jax.ShapeDtypeStruct((1, SQ, SQ), jnp.float32),
        in_specs=[
            pl.BlockSpec(memory_space=pltpu.VMEM),
            pl.BlockSpec(memory_space=pltpu.VMEM),
            pl.BlockSpec(memory_space=pl.ANY),
            pl.BlockSpec(memory_space=pl.ANY),
            pl.BlockSpec(memory_space=pltpu.VMEM),
        ],
        out_specs=pl.BlockSpec(memory_space=pltpu.VMEM),
        scratch_shapes=[
            pltpu.VMEM((KV_USED, H_PER, DH), BF),
            pltpu.VMEM((KV_USED, H_PER, DH), BF),
            pltpu.VMEM((SQ, SQ), BF),
            pltpu.VMEM((SQ, SQ), BF),
            pltpu.VMEM((SQ // 4, NH, DH), jnp.float32),
            pltpu.VMEM((SQ, NH, DH), BF),
            pltpu.VMEM((SQ, NH, DH), BF),
            pltpu.VMEM((SQ, H_PER, DH), BF),
            pltpu.VMEM((N_DEV - 1, QBLK, SQ), BF),
            pltpu.VMEM((N_DEV - 1, QBLK, SQ), BF),
            pltpu.VMEM((N_DEV, QBLK, SQ), BF),
            pltpu.SemaphoreType.DMA((8,)),
            pltpu.SemaphoreType.DMA((4,)),
            pltpu.SemaphoreType.DMA((4,)),
            pltpu.SemaphoreType.DMA((2,)),
            pltpu.SemaphoreType.DMA((3,)),
            pltpu.SemaphoreType.DMA((3,)),
            pltpu.SemaphoreType.DMA((3,)),
            pltpu.SemaphoreType.DMA((3,)),
            pltpu.SemaphoreType.DMA((3,)),
            pltpu.SemaphoreType.DMA((3,)),
        ],
        compiler_params=pltpu.CompilerParams(
            collective_id=0, vmem_limit_bytes=60 * 1024 * 1024),
    )(x, Wq, K_ext, V_ext, Wo)


# device time: 137898 ns/iter; 2.5858x vs baseline; 1.0212x over previous
import jax
import jax.numpy as jnp
from jax import lax
from jax.experimental import pallas as pl
from jax.experimental.pallas import tpu as pltpu

N_DEV = 4
SQ = 1024
H_PER = 8
NH = 32
DH = 128
WIN = 128
KV_USED = SQ + WIN
KW = 512
SCALE = 0.08838834764831843
QBLK = 256
BF = jnp.bfloat16
MESH = pl.DeviceIdType.MESH

P1 = {1: (0, 640), 2: (128, 896), 3: (384, 1024)}
P2 = {1: [(640, 1024)], 2: [(0, 128), (896, 1024)], 3: [(0, 384)]}


def kernel(x, Wq, K_ext, V_ext, Wo):
    def body(x_ref, wq_ref, k_ref, v_ref, wo_ref, out_ref,
             kbuf, vbuf, q_scr, wo16, tmp, kstage, vstage, relay,
             rs_stage, rs_buf, ag_buf,
             send_sems, recv_sems, local_sems, relay_sems,
             rs_send, rs_recv, ag_send, ag_recv, ag2_send, ag2_recv):
        i = lax.axis_index("i")
        right = lax.rem(i + 1, N_DEV)
        left = lax.rem(i - 1 + N_DEV, N_DEV)

        barrier = pltpu.get_barrier_semaphore()
        for off in (1, 2, 3):
            pl.semaphore_signal(
                barrier, inc=1,
                device_id=(lax.rem(i + off, N_DEV),),
                device_id_type=MESH)
        pl.semaphore_wait(barrier, N_DEV - 1)

        sem_idx = iter(range(14))

        def send(st, j, lo, hi, dst, rsem):
            return pltpu.make_async_remote_copy(
                src_ref=st.at[lo:hi, 8 * j:8 * (j + 1), :],
                dst_ref=dst.at[lo:hi],
                send_sem=send_sems.at[next(sem_idx)],
                recv_sem=recv_sems.at[rsem],
                device_id=(j,), device_id_type=MESH)

        def send_relay(st, lo, hi, via, ridx):
            return pltpu.make_async_remote_copy(
                src_ref=st.at[lo:hi, 16:24, :],
                dst_ref=relay.at[lo:hi],
                send_sem=send_sems.at[next(sem_idx)],
                recv_sem=relay_sems.at[ridx],
                device_id=(via,), device_id_type=MESH)

        k_w1 = [send_relay(kstage, *P1[2], via=1, ridx=0),
                send(kstage, 1, *P1[1], dst=kbuf, rsem=0),
                send(kstage, 3, *P1[3], dst=kbuf, rsem=0)]
        v_w1 = [send_relay(vstage, *P1[2], via=3, ridx=0),
                send(vstage, 1, *P1[1], dst=vbuf, rsem=1),
                send(vstage, 3, *P1[3], dst=vbuf, rsem=1)]
        w2 = [send(kstage, 1, *P2[1][0], dst=kbuf, rsem=2),
              send(vstage, 1, *P2[1][0], dst=vbuf, rsem=3),
              send(kstage, 3, *P2[3][0], dst=kbuf, rsem=2),
              send(vstage, 3, *P2[3][0], dst=vbuf, rsem=3),
              send_relay(kstage, *P2[2][0], via=1, ridx=1),
              send_relay(kstage, *P2[2][1], via=1, ridx=2),
              send_relay(vstage, *P2[2][0], via=3, ridx=1),
              send_relay(vstage, *P2[2][1], via=3, ridx=2)]

        def fwd(dst, lo, hi, sidx, rsem):
            return pltpu.make_async_remote_copy(
                src_ref=relay.at[lo:hi], dst_ref=dst.at[lo:hi],
                send_sem=send_sems.at[sidx], recv_sem=recv_sems.at[rsem],
                device_id=(2,), device_id_type=MESH)

        fwd_k = [fwd(kbuf, *P1[2], sidx=8, rsem=0),
                 fwd(kbuf, *P2[2][0], sidx=9, rsem=2),
                 fwd(kbuf, *P2[2][1], sidx=10, rsem=4)]
        fwd_v = [fwd(vbuf, *P1[2], sidx=8, rsem=1),
                 fwd(vbuf, *P2[2][0], sidx=9, rsem=3),
                 fwd(vbuf, *P2[2][1], sidx=10, rsem=5)]

        def recv_only(dst, lo, hi, rsem, dsem=13):
            return pltpu.make_async_remote_copy(
                src_ref=dst.at[lo:hi], dst_ref=dst.at[lo:hi],
                send_sem=send_sems.at[dsem], recv_sem=recv_sems.at[rsem],
                device_id=(0,), device_id_type=MESH)

        def relay_recv(lo, hi, ridx):
            return pltpu.make_async_remote_copy(
                src_ref=relay.at[lo:hi], dst_ref=relay.at[lo:hi],
                send_sem=send_sems.at[13], recv_sem=relay_sems.at[ridx],
                device_id=(0,), device_id_type=MESH)

        rr = [relay_recv(*P1[2], ridx=0),
              relay_recv(*P2[2][0], ridx=1),
              relay_recv(*P2[2][1], ridx=2)]

        r_p1 = {j: (recv_only(kbuf, *P1[j], rsem=0),
                    recv_only(vbuf, *P1[j], rsem=1)) for j in (1, 2, 3)}
        r_p2 = {j: [(recv_only(kbuf, lo, hi, rsem=2 + 2 * n),
                     recv_only(vbuf, lo, hi, rsem=3 + 2 * n))
                    for n, (lo, hi) in enumerate(P2[j])]
                for j in (1, 2, 3)}

        s1_sends = []
        for n, j in enumerate((0, 2, 3)):
            s1_sends.append(pltpu.make_async_remote_copy(
                src_ref=kstage.at[0:WIN, 8 * j:8 * (j + 1), :],
                dst_ref=kbuf.at[SQ:KV_USED],
                send_sem=send_sems.at[n],
                recv_sem=recv_sems.at[6],
                device_id=(j,), device_id_type=MESH))
            s1_sends.append(pltpu.make_async_remote_copy(
                src_ref=vstage.at[0:WIN, 8 * j:8 * (j + 1), :],
                dst_ref=vbuf.at[SQ:KV_USED],
                send_sem=send_sems.at[3 + n],
                recv_sem=recv_sems.at[7],
                device_id=(j,), device_id_type=MESH))
        r_k1 = recv_only(kbuf, SQ, KV_USED, rsem=6)
        r_v1 = recv_only(vbuf, SQ, KV_USED, rsem=7)

        QTR = SQ // 4

        @pl.when(i == 0)
        def _():
            for q in range(4):
                cp = pltpu.make_async_copy(
                    k_ref.at[0, q * QTR:(q + 1) * QTR], tmp,
                    local_sems.at[q % 2])
                cp.start()
                cp.wait()
                kstage[q * QTR:(q + 1) * QTR] = tmp[...].astype(BF)
            for s in k_w1:
                s.start()
            kbuf[0:SQ] = kstage[:, 0:H_PER, :]
            for q in range(4):
                cp = pltpu.make_async_copy(
                    v_ref.at[0, q * QTR:(q + 1) * QTR], tmp,
                    local_sems.at[2 + q % 2])
                cp.start()
                cp.wait()
                vstage[q * QTR:(q + 1) * QTR] = tmp[...].astype(BF)
            for s in v_w1:
                s.start()
            vbuf[0:SQ] = vstage[:, 0:H_PER, :]
            for s in w2:
                s.start()

        @pl.when(i == 1)
        def _():
            h1 = pltpu.make_async_copy(
                k_ref.at[0, 0:WIN], tmp.at[0:WIN], local_sems.at[0])
            h1.start()
            h1.wait()
            kstage[0:WIN] = tmp[0:WIN].astype(BF)
            h2 = pltpu.make_async_copy(
                v_ref.at[0, 0:WIN], tmp.at[WIN:2 * WIN], local_sems.at[1])
            h2.start()
            h2.wait()
            vstage[0:WIN] = tmp[WIN:2 * WIN].astype(BF)
            for s in s1_sends:
                s.start()
            kbuf[SQ:KV_USED] = kstage[0:WIN, H_PER:2 * H_PER, :]
            vbuf[SQ:KV_USED] = vstage[0:WIN, H_PER:2 * H_PER, :]

        q_scr[...] = (jnp.dot(
            x_ref[0].astype(BF), wq_ref[...].astype(BF),
            preferred_element_type=jnp.float32) * SCALE).astype(BF)
        wo16[...] = wo_ref[...].astype(BF)

        @pl.when(i == 1)
        def _():
            rr[0].wait_recv()
            fwd_k[0].start()

        @pl.when(i == 3)
        def _():
            rr[0].wait_recv()
            fwd_v[0].start()

        @pl.when(i != 1)
        def _():
            r_k1.wait_recv()
            r_v1.wait_recv()

        rs_rdmas = []
        for t in range(N_DEV):
            if t == 0:
                for j in (1, 2, 3):
                    @pl.when(i == j)
                    def _(j=j):
                        r_p1[j][0].wait_recv()
                        r_p1[j][1].wait_recv()
            if t == 2:
                @pl.when(i == 1)
                def _():
                    rr[1].wait_recv()
                    fwd_k[1].start()
                    rr[2].wait_recv()
                    fwd_k[2].start()

                @pl.when(i == 3)
                def _():
                    rr[1].wait_recv()
                    fwd_v[1].start()
                    rr[2].wait_recv()
                    fwd_v[2].start()

                for j in (1, 2, 3):
                    @pl.when(i == j)
                    def _(j=j):
                        for rk, rv in r_p2[j]:
                            rk.wait_recv()
                            rv.wait_recv()

            cb = lax.rem(i - t + 2 * N_DEV, N_DEV)
            r0 = cb * QBLK
            ks = jnp.minimum(jnp.maximum(r0 - WIN, 0), KV_USED - KW)
            rows = r0 + lax.broadcasted_iota(jnp.int32, (QBLK, KW), 0)
            cols = ks + lax.broadcasted_iota(jnp.int32, (QBLK, KW), 1)
            mask = jnp.abs(rows - cols) <= WIN
            acc = jnp.zeros((QBLK, SQ), jnp.float32)
            for h in range(H_PER):
                qh = q_scr[pl.ds(r0, QBLK), h * DH:(h + 1) * DH]
                kh = kbuf[pl.ds(ks, KW), h, :]
                vh = vbuf[pl.ds(ks, KW), h, :]
                s = lax.dot_general(
                    qh, kh, (((1,), (1,)), ((), ())),
                    preferred_element_type=jnp.float32)
                s = jnp.where(mask, s, -1e9)
                m = jnp.max(s, axis=1, keepdims=True)
                e = jnp.exp(s - m)
                den = jnp.sum(e, axis=1, keepdims=True)
                ctx = lax.dot_general(
                    e.astype(BF), vh, (((1,), (0,)), ((), ())),
                    preferred_element_type=jnp.float32) / den
                acc = acc + lax.dot_general(
                    ctx.astype(BF), wo16[h * DH:(h + 1) * DH, :],
                    (((1,), (0,)), ((), ())),
                    preferred_element_type=jnp.float32)
            out_ref[0, pl.ds(r0, QBLK), :] = acc

            if t == 0:
                rs_stage[0] = acc.astype(BF)
            else:
                rs_rdmas[t - 1].wait_recv()
                red = acc + rs_buf[t - 1].astype(jnp.float32)
                if t < N_DEV - 1:
                    rs_stage[t] = red.astype(BF)
                else:
                    out_ref[0, pl.ds(r0, QBLK), :] = red
                    ag_buf[cb] = red.astype(BF)
            if t < N_DEV - 1:
                rdma = pltpu.make_async_remote_copy(
                    src_ref=rs_stage.at[t], dst_ref=rs_buf.at[t],
                    send_sem=rs_send.at[t], recv_sem=rs_recv.at[t],
                    device_id=(right,), device_id_type=MESH)
                rdma.start()
                rs_rdmas.append(rdma)

        HC = SQ // 2
        ag_rdmas = []
        for h in range(N_DEV - 1):
            cw_s = lax.rem(i + 1 - h + 2 * N_DEV, N_DEV)
            cw_r = lax.rem(i - h + 2 * N_DEV, N_DEV)
            ccw_s = lax.rem(i + 1 + h, N_DEV)
            ccw_r = lax.rem(i + 2 + h, N_DEV)
            cw = pltpu.make_async_remote_copy(
                src_ref=ag_buf.at[cw_s, :, 0:HC],
                dst_ref=ag_buf.at[cw_s, :, 0:HC],
                send_sem=ag_send.at[h], recv_sem=ag_recv.at[h],
                device_id=(right,), device_id_type=MESH)
            ccw = pltpu.make_async_remote_copy(
                src_ref=ag_buf.at[ccw_s, :, HC:SQ],
                dst_ref=ag_buf.at[ccw_s, :, HC:SQ],
                send_sem=ag2_send.at[h], recv_sem=ag2_recv.at[h],
                device_id=(left,), device_id_type=MESH)
            cw.start()
            ccw.start()
            cw.wait_recv()
            ccw.wait_recv()
            out_ref[0, pl.ds(cw_r * QBLK, QBLK), 0:HC] = (
                ag_buf[cw_r, :, 0:HC].astype(jnp.float32))
            out_ref[0, pl.ds(ccw_r * QBLK, QBLK), HC:SQ] = (
                ag_buf[ccw_r, :, HC:SQ].astype(jnp.float32))
            ag_rdmas.append(cw)
            ag_rdmas.append(ccw)

        for r in rs_rdmas + ag_rdmas:
            r.wait_send()

        @pl.when(i == 0)
        def _():
            for s in k_w1 + v_w1 + w2:
                s.wait_send()

        @pl.when(i == 1)
        def _():
            for s in s1_sends + fwd_k:
                s.wait_send()

        @pl.when(i == 3)
        def _():
            for s in fwd_v:
                s.wait_send()

    return pl.pallas_call(
        body,
        out_shape=jax.ShapeDtypeStruct((1, SQ, SQ), jnp.float32),
        in_specs=[
            pl.BlockSpec(memory_space=pltpu.VMEM),
            pl.BlockSpec(memory_space=pltpu.VMEM),
            pl.BlockSpec(memory_space=pl.ANY),
            pl.BlockSpec(memory_space=pl.ANY),
            pl.BlockSpec(memory_space=pltpu.VMEM),
        ],
        out_specs=pl.BlockSpec(memory_space=pltpu.VMEM),
        scratch_shapes=[
            pltpu.VMEM((KV_USED, H_PER, DH), BF),
            pltpu.VMEM((KV_USED, H_PER, DH), BF),
            pltpu.VMEM((SQ, SQ), BF),
            pltpu.VMEM((SQ, SQ), BF),
            pltpu.VMEM((SQ // 4, NH, DH), jnp.float32),
            pltpu.VMEM((SQ, NH, DH), BF),
            pltpu.VMEM((SQ, NH, DH), BF),
            pltpu.VMEM((SQ, H_PER, DH), BF),
            pltpu.VMEM((N_DEV - 1, QBLK, SQ), BF),
            pltpu.VMEM((N_DEV - 1, QBLK, SQ), BF),
            pltpu.VMEM((N_DEV, QBLK, SQ), BF),
            pltpu.SemaphoreType.DMA((14,)),
            pltpu.SemaphoreType.DMA((8,)),
            pltpu.SemaphoreType.DMA((4,)),
            pltpu.SemaphoreType.DMA((3,)),
            pltpu.SemaphoreType.DMA((3,)),
            pltpu.SemaphoreType.DMA((3,)),
            pltpu.SemaphoreType.DMA((3,)),
            pltpu.SemaphoreType.DMA((3,)),
            pltpu.SemaphoreType.DMA((3,)),
            pltpu.SemaphoreType.DMA((3,)),
        ],
        compiler_params=pltpu.CompilerParams(
            collective_id=0, vmem_limit_bytes=60 * 1024 * 1024),
    )(x, Wq, K_ext, V_ext, Wo)
